# Initial kernel scaffold; baseline (speedup 1.0000x reference)
import jax
import jax.numpy as jnp
from jax.experimental import pallas as pl


def kernel(x, edge_index, batch, W_phi, b_phi, W_mlp, b_mlp):
    raise NotImplementedError("write your pallas kernel here")


# speedup vs baseline: 31.5956x; 31.5956x over previous
import functools

import numpy as np
import jax
import jax.numpy as jnp
from jax import lax
from jax.experimental import pallas as pl
from jax.experimental.pallas import tpu as pltpu
from jax.experimental.pallas import tpu_sc as plsc

_G = 64


def _make_sc_counts(N, E, G):
    info = plsc.get_sparse_core_info()
    NC, NS, L = info.num_cores, info.num_subcores, info.num_lanes
    assert NC == 2, NC
    assert E % (NC * NS * L) == 0, (E, NC, NS, L)
    EPT = E // (NC * NS)
    NV = EPT // L
    ROWS = 8 * (-(-EPT // (128 * 8)))
    GN = G * N
    ACCP = -(-(GN + N) // (NS * 4096)) * (NS * 4096)
    SLICE = ACCP // NS
    CH = 4096
    assert SLICE % CH == 0 and CH % L == 0
    mesh = plsc.VectorSubcoreMesh(core_axis_name="c", subcore_axis_name="s")

    @functools.partial(
        pl.kernel,
        out_type=jax.ShapeDtypeStruct((NC * ACCP,), jnp.float32),
        mesh=mesh,
        compiler_params=pltpu.CompilerParams(needs_layout_passes=False),
        scratch_types=[
            pltpu.VMEM((N,), jnp.int32),
            pltpu.VMEM((EPT,), jnp.int32),
            pltpu.VMEM((EPT,), jnp.int32),
            pltpu.VMEM((ROWS, 128), jnp.int32),
            pltpu.VMEM((ROWS, 128), jnp.int32),
            pltpu.VMEM((ROWS, 128), jnp.float32),
            pltpu.VMEM((CH,), jnp.float32),
            pltpu.VMEM_SHARED((ACCP,), jnp.float32),
        ],
    )
    def sc_counts(edge_hbm, batch_hbm, vals_hbm, out_hbm,
                  batch_v, src_v, dst_v, idx2_v, idx1_v, val_v, zb_v, acc_sh):
        c = lax.axis_index("c")
        s = lax.axis_index("s")
        ebase = (c * NS + s) * EPT

        def zfill(i, carry):
            zb_v[pl.ds(i * L, L)] = jnp.zeros((L,), jnp.float32)
            return carry

        lax.fori_loop(0, CH // L, zfill, 0)
        for j in range(SLICE // CH):
            pltpu.sync_copy(zb_v, acc_sh.at[pl.ds(s * SLICE + j * CH, CH)])

        pltpu.sync_copy(batch_hbm, batch_v)
        pltpu.sync_copy(edge_hbm.at[pl.ds(ebase, EPT)], src_v)
        pltpu.sync_copy(edge_hbm.at[pl.ds(E + ebase, EPT)], dst_v)
        pltpu.sync_copy(vals_hbm, val_v)

        for t in range(NV, ROWS * 8):
            idx2_v[t // 8, pl.ds((t % 8) * L, L)] = jnp.zeros((L,), jnp.int32)
            idx1_v[t // 8, pl.ds((t % 8) * L, L)] = jnp.zeros((L,), jnp.int32)

        def body(i, carry):
            sv = src_v[pl.ds(i * L, L)]
            dv = dst_v[pl.ds(i * L, L)]
            gv = plsc.load_gather(batch_v, [sv])
            idx2_v[i // 8, pl.ds((i % 8) * L, L)] = gv * N + dv
            idx1_v[i // 8, pl.ds((i % 8) * L, L)] = sv + GN
            return carry

        lax.fori_loop(0, NV, body, 0)

        plsc.subcore_barrier()

        def srow(r, carry):
            pltpu.sync_copy(val_v.at[r], acc_sh.at[idx2_v.at[r]], add=True)
            pltpu.sync_copy(val_v.at[r], acc_sh.at[idx1_v.at[r]], add=True)
            return carry

        lax.fori_loop(0, ROWS, srow, 0)
        plsc.subcore_barrier()

        for j in range(SLICE // CH):
            pltpu.sync_copy(acc_sh.at[pl.ds(s * SLICE + j * CH, CH)], zb_v)
            pltpu.sync_copy(zb_v, out_hbm.at[pl.ds(c * ACCP + s * SLICE + j * CH, CH)])

    return sc_counts, EPT, ROWS, ACCP, NC


def _tc_dense(c2h, c1h, batch2d, x, W_phi, b_phi2d, W_mlp, b_mlp2d, G):
    N, D = x.shape

    def body(c2h_ref, c1h_ref, batch_ref, x_ref, wphi_ref, bphi_ref,
             wmlp_ref, bmlp_ref, o_ref):
        C2 = c2h_ref[0] + c2h_ref[1]
        c1 = c1h_ref[0:1, :] + c1h_ref[1:2, :]
        gids = lax.broadcasted_iota(jnp.int32, (G, N), 0)
        B1 = jnp.where(gids == batch_ref[...], 1.0, 0.0)
        M1 = B1 * c1
        S1 = jnp.dot(M1, x_ref[...], preferred_element_type=jnp.float32)
        S2 = jnp.dot(C2, x_ref[...], preferred_element_type=jnp.float32)
        cnt = jnp.sum(C2, axis=1, keepdims=True)
        Pcat = jnp.concatenate([S1, S2], axis=1)
        pooled = lax.dot_general(Pcat, wphi_ref[...],
                                 (((1,), (1,)), ((), ())),
                                 preferred_element_type=jnp.float32)
        pooled = pooled + cnt * bphi_ref[...]
        out = lax.dot_general(pooled, wmlp_ref[...],
                              (((1,), (1,)), ((), ())),
                              preferred_element_type=jnp.float32)
        o_ref[...] = out + bmlp_ref[...]

    return pl.pallas_call(
        body,
        out_shape=jax.ShapeDtypeStruct((G, D), jnp.float32),
    )(c2h, c1h, batch2d, x, W_phi, b_phi2d, W_mlp, b_mlp2d)


def kernel(x, edge_index, batch, W_phi, b_phi, W_mlp, b_mlp):
    N, D = x.shape
    E = edge_index.shape[1]
    G = _G
    GN = G * N
    sc_counts, EPT, ROWS, ACCP, NC = _make_sc_counts(N, E, G)

    ent = np.arange(ROWS * 128)
    vals = jnp.asarray((ent < EPT).astype(np.float32).reshape(ROWS, 128))

    acc = sc_counts(edge_index.reshape(-1), batch, vals)
    acc = acc.reshape(NC, ACCP)
    c2h = acc[:, :GN].reshape(NC, G, N)
    c1h = acc[:, GN:GN + N]
    return _tc_dense(c2h, c1h, batch.reshape(1, N), x, W_phi,
                     b_phi.reshape(1, D), W_mlp, b_mlp.reshape(1, D), G)


# speedup vs baseline: 39.8344x; 1.2608x over previous
import functools

import numpy as np
import jax
import jax.numpy as jnp
from jax import lax
from jax.experimental import pallas as pl
from jax.experimental.pallas import tpu as pltpu
from jax.experimental.pallas import tpu_sc as plsc

_G = 64


def _make_sc_counts(N, E, G):
    info = plsc.get_sparse_core_info()
    NC, NS, L = info.num_cores, info.num_subcores, info.num_lanes
    assert NC == 2, NC
    assert E % (NC * NS * L) == 0, (E, NC, NS, L)
    assert G % NS == 0 and N % L == 0
    EPT = E // (NC * NS)
    NV = EPT // L
    ROWS = 8 * (-(-EPT // (128 * 8)))
    GN = G * N
    GPT = G // NS
    mesh = plsc.VectorSubcoreMesh(core_axis_name="c", subcore_axis_name="s")

    @functools.partial(
        pl.kernel,
        out_type=[
            jax.ShapeDtypeStruct((NC, G, N), jnp.float32),
            jax.ShapeDtypeStruct((NC, N), jnp.float32),
        ],
        mesh=mesh,
        compiler_params=pltpu.CompilerParams(needs_layout_passes=False),
        scratch_types=[
            pltpu.VMEM((N,), jnp.int32),
            pltpu.VMEM((EPT,), jnp.int32),
            pltpu.VMEM((EPT,), jnp.int32),
            pltpu.VMEM((ROWS, 128), jnp.int32),
            pltpu.VMEM((ROWS, 128), jnp.int32),
            pltpu.VMEM((ROWS, 128), jnp.float32),
            pltpu.VMEM((N,), jnp.float32),
            pltpu.VMEM_SHARED((GN + N,), jnp.float32),
        ],
    )
    def sc_counts(edge_hbm, batch_hbm, vals_hbm, out2_hbm, out1_hbm,
                  batch_v, src_v, dst_v, idx2_v, idx1_v, val_v, row_v, acc_sh):
        c = lax.axis_index("c")
        s = lax.axis_index("s")
        ebase = (c * NS + s) * EPT

        with jax.named_scope("zero"):
            def zfill(i, carry):
                row_v[pl.ds(i * L, L)] = jnp.zeros((L,), jnp.float32)
                return carry

            lax.fori_loop(0, N // L, zfill, 0)
            for r in range(GPT):
                pltpu.sync_copy(row_v, acc_sh.at[pl.ds((s * GPT + r) * N, N)])

            @pl.when(s == 0)
            def _():
                pltpu.sync_copy(row_v, acc_sh.at[pl.ds(GN, N)])

        with jax.named_scope("stage"):
            pltpu.sync_copy(batch_hbm, batch_v)
            pltpu.sync_copy(edge_hbm.at[pl.ds(ebase, EPT)], src_v)
            pltpu.sync_copy(edge_hbm.at[pl.ds(E + ebase, EPT)], dst_v)
            pltpu.sync_copy(vals_hbm, val_v)

        with jax.named_scope("build"):
            for t in range(NV, ROWS * 8):
                idx2_v[t // 8, pl.ds((t % 8) * L, L)] = jnp.zeros((L,), jnp.int32)
                idx1_v[t // 8, pl.ds((t % 8) * L, L)] = jnp.zeros((L,), jnp.int32)

            def body(i, carry):
                sv = src_v[pl.ds(i * L, L)]
                dv = dst_v[pl.ds(i * L, L)]
                gv = plsc.load_gather(batch_v, [sv])
                idx2_v[i // 8, pl.ds((i % 8) * L, L)] = gv * N + dv
                idx1_v[i // 8, pl.ds((i % 8) * L, L)] = sv + GN
                return carry

            lax.fori_loop(0, NV, body, 0)

        plsc.subcore_barrier()

        with jax.named_scope("scatter"):
            def srow(r, carry):
                pltpu.sync_copy(val_v.at[r], acc_sh.at[idx2_v.at[r]], add=True)
                pltpu.sync_copy(val_v.at[r], acc_sh.at[idx1_v.at[r]], add=True)
                return carry

            lax.fori_loop(0, ROWS, srow, 0)
        plsc.subcore_barrier()

        with jax.named_scope("writeback"):
            for r in range(GPT):
                g = s * GPT + r
                pltpu.sync_copy(acc_sh.at[pl.ds(g * N, N)], row_v)
                pltpu.sync_copy(row_v, out2_hbm.at[c, g])

            @pl.when(s == 0)
            def _():
                pltpu.sync_copy(acc_sh.at[pl.ds(GN, N)], row_v)
                pltpu.sync_copy(row_v, out1_hbm.at[c])

    return sc_counts, EPT, ROWS, NC


def _tc_dense(c2h, c1h, batch2d, x, W_phi, b_phi2d, W_mlp, b_mlp2d, G):
    N, D = x.shape

    def body(c2h_ref, c1h_ref, batch_ref, x_ref, wphi_ref, bphi_ref,
             wmlp_ref, bmlp_ref, o_ref):
        C2 = c2h_ref[0] + c2h_ref[1]
        c1 = c1h_ref[0:1, :] + c1h_ref[1:2, :]
        gids = lax.broadcasted_iota(jnp.int32, (G, N), 0)
        B1 = jnp.where(gids == batch_ref[...], 1.0, 0.0)
        M1 = B1 * c1
        S1 = jnp.dot(M1, x_ref[...], preferred_element_type=jnp.float32)
        S2 = jnp.dot(C2, x_ref[...], preferred_element_type=jnp.float32)
        cnt = jnp.sum(C2, axis=1, keepdims=True)
        Pcat = jnp.concatenate([S1, S2], axis=1)
        pooled = lax.dot_general(Pcat, wphi_ref[...],
                                 (((1,), (1,)), ((), ())),
                                 preferred_element_type=jnp.float32)
        pooled = pooled + cnt * bphi_ref[...]
        out = lax.dot_general(pooled, wmlp_ref[...],
                              (((1,), (1,)), ((), ())),
                              preferred_element_type=jnp.float32)
        o_ref[...] = out + bmlp_ref[...]

    return pl.pallas_call(
        body,
        out_shape=jax.ShapeDtypeStruct((G, D), jnp.float32),
    )(c2h, c1h, batch2d, x, W_phi, b_phi2d, W_mlp, b_mlp2d)


def kernel(x, edge_index, batch, W_phi, b_phi, W_mlp, b_mlp):
    N, D = x.shape
    E = edge_index.shape[1]
    G = _G
    sc_counts, EPT, ROWS, NC = _make_sc_counts(N, E, G)

    ent = np.arange(ROWS * 128)
    vals = jnp.asarray((ent < EPT).astype(np.float32).reshape(ROWS, 128))

    c2h, c1h = sc_counts(edge_index.reshape(-1), batch, vals)
    return _tc_dense(c2h, c1h, batch.reshape(1, N), x, W_phi,
                     b_phi.reshape(1, D), W_mlp, b_mlp.reshape(1, D), G)


# speedup vs baseline: 42.6300x; 1.0702x over previous
import functools

import numpy as np
import jax
import jax.numpy as jnp
from jax import lax
from jax.experimental import pallas as pl
from jax.experimental.pallas import tpu as pltpu
from jax.experimental.pallas import tpu_sc as plsc

_G = 64


def _make_sc_counts(N, E, G):
    info = plsc.get_sparse_core_info()
    NC, NS, L = info.num_cores, info.num_subcores, info.num_lanes
    assert NC == 2, NC
    assert E % (NC * NS * L) == 0, (E, NC, NS, L)
    assert G % NS == 0 and N % L == 0
    EPT = E // (NC * NS)
    NV = EPT // L
    ROWS = 8 * (-(-EPT // (128 * 8)))
    GN = G * N
    GPT = G // NS
    NP = -(-N // (NS * L)) * (NS * L)
    W = NP // NS
    WV = W // L
    mesh = plsc.VectorSubcoreMesh(core_axis_name="c", subcore_axis_name="s")

    @functools.partial(
        pl.kernel,
        out_type=[
            jax.ShapeDtypeStruct((NC, G, N), jnp.float32),
            jax.ShapeDtypeStruct((NC, N), jnp.float32),
        ],
        mesh=mesh,
        compiler_params=pltpu.CompilerParams(needs_layout_passes=False),
        scratch_types=[
            pltpu.VMEM((N,), jnp.int32),
            pltpu.VMEM((EPT,), jnp.int32),
            pltpu.VMEM((EPT,), jnp.int32),
            pltpu.VMEM((ROWS, 128), jnp.int32),
            pltpu.VMEM((ROWS, 128), jnp.float32),
            pltpu.VMEM((N,), jnp.float32),
            pltpu.VMEM((NP,), jnp.float32),
            pltpu.VMEM_SHARED((GN + NP,), jnp.float32),
            pltpu.VMEM_SHARED((NS, NP), jnp.float32),
        ],
    )
    def sc_counts(edge_hbm, batch_hbm, vals_hbm, out2_hbm, out1_hbm,
                  batch_v, src_v, dst_v, idx2_v, val_v, row_v, c1_v,
                  acc_sh, stage_sh):
        c = lax.axis_index("c")
        s = lax.axis_index("s")
        ebase = (c * NS + s) * EPT

        with jax.named_scope("zero"):
            def zfill(i, carry):
                row_v[pl.ds(i * L, L)] = jnp.zeros((L,), jnp.float32)
                c1_v[pl.ds(i * L, L)] = jnp.zeros((L,), jnp.float32)
                return carry

            lax.fori_loop(0, N // L, zfill, 0)
            for t in range((NP - N) // L):
                c1_v[pl.ds(N + t * L, L)] = jnp.zeros((L,), jnp.float32)
            for r in range(GPT):
                pltpu.sync_copy(row_v, acc_sh.at[pl.ds((s * GPT + r) * N, N)])

        with jax.named_scope("stage"):
            pltpu.sync_copy(batch_hbm, batch_v)
            pltpu.sync_copy(edge_hbm.at[pl.ds(ebase, EPT)], src_v)
            pltpu.sync_copy(edge_hbm.at[pl.ds(E + ebase, EPT)], dst_v)
            pltpu.sync_copy(vals_hbm, val_v)

        with jax.named_scope("build"):
            for t in range(NV, ROWS * 8):
                idx2_v[t // 8, pl.ds((t % 8) * L, L)] = jnp.zeros((L,), jnp.int32)

            def body(i, carry):
                sv = src_v[pl.ds(i * L, L)]
                dv = dst_v[pl.ds(i * L, L)]
                gv = plsc.load_gather(batch_v, [sv])
                idx2_v[i // 8, pl.ds((i % 8) * L, L)] = gv * N + dv
                plsc.addupdate_scatter(c1_v, [sv], jnp.ones((L,), jnp.float32))
                return carry

            lax.fori_loop(0, NV, body, 0)

        with jax.named_scope("publish"):
            pltpu.sync_copy(c1_v, stage_sh.at[s])

        plsc.subcore_barrier()

        with jax.named_scope("scatter"):
            def srow(r, carry):
                pltpu.sync_copy(val_v.at[r], acc_sh.at[idx2_v.at[r]], add=True)
                return carry

            lax.fori_loop(0, ROWS, srow, 0)

        with jax.named_scope("reduce"):
            for k in range(NS):
                pltpu.sync_copy(stage_sh.at[k, pl.ds(s * W, W)],
                                c1_v.at[pl.ds(k * W, W)])

            def rsum(v, carry):
                a = c1_v[pl.ds(v * L, L)]
                for k in range(1, NS):
                    a = a + c1_v[pl.ds(k * W + v * L, L)]
                row_v[pl.ds(v * L, L)] = a
                return carry

            lax.fori_loop(0, WV, rsum, 0)
            pltpu.sync_copy(row_v.at[pl.ds(0, W)],
                            acc_sh.at[pl.ds(GN + s * W, W)])

        plsc.subcore_barrier()

        with jax.named_scope("writeback"):
            for r in range(GPT):
                g = s * GPT + r
                pltpu.sync_copy(acc_sh.at[pl.ds(g * N, N)], row_v)
                pltpu.sync_copy(row_v, out2_hbm.at[c, g])

            @pl.when(s == 0)
            def _():
                pltpu.sync_copy(acc_sh.at[pl.ds(GN, N)], row_v)
                pltpu.sync_copy(row_v, out1_hbm.at[c])

    return sc_counts, EPT, ROWS, NC


def _tc_dense(c2h, c1h, batch2d, x, W_phi, b_phi2d, W_mlp, b_mlp2d, G):
    N, D = x.shape

    def body(c2h_ref, c1h_ref, batch_ref, x_ref, wphi_ref, bphi_ref,
             wmlp_ref, bmlp_ref, o_ref):
        C2 = c2h_ref[0] + c2h_ref[1]
        c1 = c1h_ref[0:1, :] + c1h_ref[1:2, :]
        gids = lax.broadcasted_iota(jnp.int32, (G, N), 0)
        B1 = jnp.where(gids == batch_ref[...], 1.0, 0.0)
        M1 = B1 * c1
        S1 = jnp.dot(M1, x_ref[...], preferred_element_type=jnp.float32)
        S2 = jnp.dot(C2, x_ref[...], preferred_element_type=jnp.float32)
        cnt = jnp.sum(C2, axis=1, keepdims=True)
        Pcat = jnp.concatenate([S1, S2], axis=1)
        pooled = lax.dot_general(Pcat, wphi_ref[...],
                                 (((1,), (1,)), ((), ())),
                                 preferred_element_type=jnp.float32)
        pooled = pooled + cnt * bphi_ref[...]
        out = lax.dot_general(pooled, wmlp_ref[...],
                              (((1,), (1,)), ((), ())),
                              preferred_element_type=jnp.float32)
        o_ref[...] = out + bmlp_ref[...]

    return pl.pallas_call(
        body,
        out_shape=jax.ShapeDtypeStruct((G, D), jnp.float32),
    )(c2h, c1h, batch2d, x, W_phi, b_phi2d, W_mlp, b_mlp2d)


def kernel(x, edge_index, batch, W_phi, b_phi, W_mlp, b_mlp):
    N, D = x.shape
    E = edge_index.shape[1]
    G = _G
    sc_counts, EPT, ROWS, NC = _make_sc_counts(N, E, G)

    ent = np.arange(ROWS * 128)
    vals = jnp.asarray((ent < EPT).astype(np.float32).reshape(ROWS, 128))

    c2h, c1h = sc_counts(edge_index.reshape(-1), batch, vals)
    return _tc_dense(c2h, c1h, batch.reshape(1, N), x, W_phi,
                     b_phi.reshape(1, D), W_mlp, b_mlp.reshape(1, D), G)


# speedup vs baseline: 49.9502x; 1.1717x over previous
import functools

import numpy as np
import jax
import jax.numpy as jnp
from jax import lax
from jax.experimental import pallas as pl
from jax.experimental.pallas import tpu as pltpu
from jax.experimental.pallas import tpu_sc as plsc

_G = 64


def _make_sc_counts(N, E, G):
    info = plsc.get_sparse_core_info()
    NC, NS, L = info.num_cores, info.num_subcores, info.num_lanes
    assert NC == 2, NC
    assert E % (NC * NS * L) == 0, (E, NC, NS, L)
    assert G % NS == 0 and N % L == 0
    EPT = E // (NC * NS)
    NV = EPT // L
    ROWS = 8 * (-(-EPT // (128 * 8)))
    GN = G * N
    GPT = G // NS
    NP = -(-N // (NS * L)) * (NS * L)
    W = NP // NS
    WV = W // L
    mesh = plsc.VectorSubcoreMesh(core_axis_name="c", subcore_axis_name="s")

    DUMP = GN + N

    @functools.partial(
        pl.kernel,
        out_type=[
            jax.ShapeDtypeStruct((NC, G, N), jnp.float32),
            jax.ShapeDtypeStruct((NC, N), jnp.float32),
        ],
        mesh=mesh,
        compiler_params=pltpu.CompilerParams(needs_layout_passes=False),
        scratch_types=[
            pltpu.VMEM((N,), jnp.int32),
            pltpu.VMEM((EPT,), jnp.int32),
            pltpu.VMEM((EPT,), jnp.int32),
            pltpu.VMEM((ROWS, 128), jnp.int32),
            pltpu.VMEM((128,), jnp.float32),
            pltpu.VMEM((N,), jnp.float32),
            pltpu.VMEM((NP,), jnp.float32),
            pltpu.VMEM_SHARED((GN + NP,), jnp.float32),
            pltpu.VMEM_SHARED((NS, NP), jnp.float32),
            pltpu.SemaphoreType.DMA,
            pltpu.SemaphoreType.DMA,
            pltpu.SemaphoreType.DMA,
        ],
    )
    def sc_counts(edge_hbm, batch_hbm, out2_hbm, out1_hbm,
                  batch_v, src_v, dst_v, idx2_v, ones_v, row_v, c1_v,
                  acc_sh, stage_sh, sem_st, sem_rd, sem_wb):
        c = lax.axis_index("c")
        s = lax.axis_index("s")
        ebase = (c * NS + s) * EPT

        with jax.named_scope("stage_start"):
            h_b = pltpu.async_copy(batch_hbm, batch_v, sem_st)
            h_s = pltpu.async_copy(edge_hbm.at[pl.ds(ebase, EPT)], src_v, sem_st)
            h_d = pltpu.async_copy(edge_hbm.at[pl.ds(E + ebase, EPT)], dst_v, sem_st)

        with jax.named_scope("zero"):
            for i in range(8):
                ones_v[pl.ds(i * L, L)] = jnp.ones((L,), jnp.float32)

            def zfill(i, carry):
                row_v[pl.ds(i * L, L)] = jnp.zeros((L,), jnp.float32)
                c1_v[pl.ds(i * L, L)] = jnp.zeros((L,), jnp.float32)
                return carry

            lax.fori_loop(0, N // L, zfill, 0)
            for t in range((NP - N) // L):
                c1_v[pl.ds(N + t * L, L)] = jnp.zeros((L,), jnp.float32)
            for r in range(GPT):
                pltpu.sync_copy(row_v, acc_sh.at[pl.ds((s * GPT + r) * N, N)])

        with jax.named_scope("stage_wait"):
            h_b.wait()
            h_s.wait()
            h_d.wait()

        with jax.named_scope("build"):
            for t in range(NV, ROWS * 8):
                idx2_v[t // 8, pl.ds((t % 8) * L, L)] = jnp.full(
                    (L,), DUMP, jnp.int32)

            def body(i, carry):
                sv = src_v[pl.ds(i * L, L)]
                dv = dst_v[pl.ds(i * L, L)]
                gv = plsc.load_gather(batch_v, [sv])
                idx2_v[i // 8, pl.ds((i % 8) * L, L)] = gv * N + dv
                plsc.addupdate_scatter(c1_v, [sv], jnp.ones((L,), jnp.float32))
                return carry

            lax.fori_loop(0, NV, body, 0)

        with jax.named_scope("publish"):
            pltpu.sync_copy(c1_v, stage_sh.at[s])

        plsc.subcore_barrier()

        with jax.named_scope("scatter"):
            def srow(r, carry):
                pltpu.sync_copy(ones_v, acc_sh.at[idx2_v.at[r]], add=True)
                return carry

            lax.fori_loop(0, ROWS, srow, 0)

        with jax.named_scope("reduce"):
            hs = [pltpu.async_copy(stage_sh.at[k, pl.ds(s * W, W)],
                                   c1_v.at[pl.ds(k * W, W)], sem_rd)
                  for k in range(NS)]
            for h in hs:
                h.wait()

            def rsum(v, carry):
                a = c1_v[pl.ds(v * L, L)]
                for k in range(1, NS):
                    a = a + c1_v[pl.ds(k * W + v * L, L)]
                row_v[pl.ds(v * L, L)] = a
                return carry

            lax.fori_loop(0, WV, rsum, 0)
            pltpu.sync_copy(row_v.at[pl.ds(0, W)],
                            acc_sh.at[pl.ds(GN + s * W, W)])

        plsc.subcore_barrier()

        with jax.named_scope("writeback"):
            for r in range(GPT):
                g = s * GPT + r
                pltpu.sync_copy(acc_sh.at[pl.ds(g * N, N)], row_v)
                pltpu.sync_copy(row_v, out2_hbm.at[c, g])

            @pl.when(s == 0)
            def _():
                pltpu.sync_copy(acc_sh.at[pl.ds(GN, N)], row_v)
                pltpu.sync_copy(row_v, out1_hbm.at[c])

    return sc_counts, EPT, ROWS, NC


def _tc_dense(c2h, c1h, batch2d, x, W_phi, b_phi2d, W_mlp, b_mlp2d, G):
    N, D = x.shape

    def body(c2h_ref, c1h_ref, batch_ref, x_ref, wphi_ref, bphi_ref,
             wmlp_ref, bmlp_ref, o_ref):
        C2 = c2h_ref[0] + c2h_ref[1]
        c1 = c1h_ref[0:1, :] + c1h_ref[1:2, :]
        gids = lax.broadcasted_iota(jnp.int32, (G, N), 0)
        B1 = jnp.where(gids == batch_ref[...], 1.0, 0.0)
        M1 = B1 * c1
        S1 = jnp.dot(M1, x_ref[...], preferred_element_type=jnp.float32)
        S2 = jnp.dot(C2, x_ref[...], preferred_element_type=jnp.float32)
        cnt = jnp.sum(C2, axis=1, keepdims=True)
        Pcat = jnp.concatenate([S1, S2], axis=1)
        pooled = lax.dot_general(Pcat, wphi_ref[...],
                                 (((1,), (1,)), ((), ())),
                                 preferred_element_type=jnp.float32)
        pooled = pooled + cnt * bphi_ref[...]
        out = lax.dot_general(pooled, wmlp_ref[...],
                              (((1,), (1,)), ((), ())),
                              preferred_element_type=jnp.float32)
        o_ref[...] = out + bmlp_ref[...]

    return pl.pallas_call(
        body,
        out_shape=jax.ShapeDtypeStruct((G, D), jnp.float32),
    )(c2h, c1h, batch2d, x, W_phi, b_phi2d, W_mlp, b_mlp2d)


def kernel(x, edge_index, batch, W_phi, b_phi, W_mlp, b_mlp):
    N, D = x.shape
    E = edge_index.shape[1]
    G = _G
    sc_counts, EPT, ROWS, NC = _make_sc_counts(N, E, G)

    c2h, c1h = sc_counts(edge_index.reshape(-1), batch)
    return _tc_dense(c2h, c1h, batch.reshape(1, N), x, W_phi,
                     b_phi.reshape(1, D), W_mlp, b_mlp.reshape(1, D), G)


# speedup vs baseline: 56.9316x; 1.1398x over previous
import functools

import numpy as np
import jax
import jax.numpy as jnp
from jax import lax
from jax.experimental import pallas as pl
from jax.experimental.pallas import tpu as pltpu
from jax.experimental.pallas import tpu_sc as plsc

_G = 64


def _make_sc_counts(N, E, G):
    info = plsc.get_sparse_core_info()
    NC, NS, L = info.num_cores, info.num_subcores, info.num_lanes
    assert NC == 2, NC
    assert E % (NC * NS * L) == 0, (E, NC, NS, L)
    assert G % NS == 0 and N % L == 0
    EPT = E // (NC * NS)
    NV = EPT // L
    ROWS = 8 * (-(-EPT // (128 * 8)))
    GN = G * N
    GPT = G // NS
    NP = -(-N // (NS * L)) * (NS * L)
    assert NP % 1024 == 0
    W = NP // NS
    WV = W // L
    mesh = plsc.VectorSubcoreMesh(core_axis_name="c", subcore_axis_name="s")

    DUMP = GN + N

    @functools.partial(
        pl.kernel,
        out_type=[
            jax.ShapeDtypeStruct((NC, G, NP), jnp.float32),
            jax.ShapeDtypeStruct((NC, NP), jnp.float32),
        ],
        mesh=mesh,
        compiler_params=pltpu.CompilerParams(needs_layout_passes=False),
        scratch_types=[
            pltpu.VMEM((N,), jnp.int32),
            pltpu.VMEM((EPT,), jnp.int32),
            pltpu.VMEM((EPT,), jnp.int32),
            pltpu.VMEM((ROWS, 128), jnp.int32),
            pltpu.VMEM((128,), jnp.float32),
            pltpu.VMEM((NP,), jnp.float32),
            pltpu.VMEM((NP,), jnp.float32),
            pltpu.VMEM_SHARED((GN + NP,), jnp.float32),
            pltpu.VMEM_SHARED((NS, NP), jnp.float32),
            pltpu.SemaphoreType.DMA,
            pltpu.SemaphoreType.DMA,
            pltpu.SemaphoreType.DMA,
            pltpu.SemaphoreType.DMA,
        ],
    )
    def sc_counts(edge_hbm, batch_hbm, out2_hbm, out1_hbm,
                  batch_v, src_v, dst_v, idx2_v, ones_v, row_v, c1_v,
                  acc_sh, stage_sh, sem_st, sem_sc, sem_rd, sem_wb):
        c = lax.axis_index("c")
        s = lax.axis_index("s")
        ebase = (c * NS + s) * EPT

        with jax.named_scope("stage_start"):
            h_b = pltpu.async_copy(batch_hbm, batch_v, sem_st)
            h_s = pltpu.async_copy(edge_hbm.at[pl.ds(ebase, EPT)], src_v, sem_st)
            h_d = pltpu.async_copy(edge_hbm.at[pl.ds(E + ebase, EPT)], dst_v, sem_st)

        with jax.named_scope("zero"):
            for i in range(8):
                ones_v[pl.ds(i * L, L)] = jnp.ones((L,), jnp.float32)

            def zfill(i, carry):
                row_v[pl.ds(i * L, L)] = jnp.zeros((L,), jnp.float32)
                return carry

            lax.fori_loop(0, N // L, zfill, 0)
            for r in range(GPT):
                pltpu.sync_copy(row_v.at[pl.ds(0, N)],
                                acc_sh.at[pl.ds((s * GPT + r) * N, N)])
            pltpu.sync_copy(acc_sh.at[pl.ds(s * GPT * N, NP)], c1_v)

        with jax.named_scope("stage_wait"):
            h_b.wait()
            h_s.wait()
            h_d.wait()

        with jax.named_scope("pad"):
            for t in range(NV, ROWS * 8):
                idx2_v[t // 8, pl.ds((t % 8) * L, L)] = jnp.full(
                    (L,), DUMP, jnp.int32)

        plsc.subcore_barrier()

        with jax.named_scope("build_scatter"):
            def body(i, carry):
                sv = src_v[pl.ds(i * L, L)]
                dv = dst_v[pl.ds(i * L, L)]
                gv = plsc.load_gather(batch_v, [sv])
                idx2_v[i // 8, pl.ds((i % 8) * L, L)] = gv * N + dv
                plsc.addupdate_scatter(c1_v, [sv], jnp.ones((L,), jnp.float32))
                return carry

            CR = 10
            K = ROWS // CR
            handles = {}
            for k in range(K):
                lax.fori_loop(k * CR * 8, min((k + 1) * CR * 8, NV), body, 0)
                if k - 2 in handles:
                    for h in handles.pop(k - 2):
                        h.wait()
                handles[k] = [
                    pltpu.async_copy(ones_v, acc_sh.at[idx2_v.at[r]], sem_sc,
                                     add=True)
                    for r in range(k * CR, (k + 1) * CR)
                ]

        with jax.named_scope("publish"):
            pltpu.sync_copy(c1_v, stage_sh.at[s])

        plsc.subcore_barrier()

        with jax.named_scope("reduce"):
            hs = [pltpu.async_copy(stage_sh.at[k, pl.ds(s * W, W)],
                                   c1_v.at[pl.ds(k * W, W)], sem_rd)
                  for k in range(NS)]
            for h in hs:
                h.wait()

            def rsum(v, carry):
                a = c1_v[pl.ds(v * L, L)]
                for k in range(1, NS):
                    a = a + c1_v[pl.ds(k * W + v * L, L)]
                row_v[pl.ds(v * L, L)] = a
                return carry

            lax.fori_loop(0, WV, rsum, 0)
            pltpu.sync_copy(row_v.at[pl.ds(0, W)],
                            acc_sh.at[pl.ds(GN + s * W, W)])

        with jax.named_scope("drain"):
            for k in sorted(handles):
                for h in handles[k]:
                    h.wait()

        plsc.subcore_barrier()

        with jax.named_scope("writeback"):
            bufs = [row_v, c1_v]
            pltpu.sync_copy(acc_sh.at[pl.ds(s * GPT * N, N)],
                            bufs[0].at[pl.ds(0, N)])
            for r in range(GPT):
                g = s * GPT + r
                if r + 1 < GPT:
                    h = pltpu.async_copy(acc_sh.at[pl.ds((g + 1) * N, N)],
                                         bufs[(r + 1) % 2].at[pl.ds(0, N)],
                                         sem_wb)
                pltpu.sync_copy(bufs[r % 2], out2_hbm.at[c, g])
                if r + 1 < GPT:
                    h.wait()

            @pl.when(s == 0)
            def _():
                pltpu.sync_copy(acc_sh.at[pl.ds(GN, NP)], row_v)
                pltpu.sync_copy(row_v, out1_hbm.at[c])

    return sc_counts, EPT, ROWS, NC


def _tc_dense(c2h, c1h, batch2d, x, W_phi, b_phi2d, W_mlp, b_mlp2d, G):
    N, D = x.shape

    def body(c2h_ref, c1h_ref, batch_ref, x_ref, wphi_ref, bphi_ref,
             wmlp_ref, bmlp_ref, o_ref):
        C2 = c2h_ref[0, :, :N] + c2h_ref[1, :, :N]
        c1 = c1h_ref[0:1, :N] + c1h_ref[1:2, :N]
        gids = lax.broadcasted_iota(jnp.int32, (G, N), 0)
        B1 = jnp.where(gids == batch_ref[...], 1.0, 0.0)
        M1 = B1 * c1
        S1 = jnp.dot(M1, x_ref[...], preferred_element_type=jnp.float32)
        S2 = jnp.dot(C2, x_ref[...], preferred_element_type=jnp.float32)
        cnt = jnp.sum(C2, axis=1, keepdims=True)
        Pcat = jnp.concatenate([S1, S2], axis=1)
        pooled = lax.dot_general(Pcat, wphi_ref[...],
                                 (((1,), (1,)), ((), ())),
                                 preferred_element_type=jnp.float32)
        pooled = pooled + cnt * bphi_ref[...]
        out = lax.dot_general(pooled, wmlp_ref[...],
                              (((1,), (1,)), ((), ())),
                              preferred_element_type=jnp.float32)
        o_ref[...] = out + bmlp_ref[...]

    return pl.pallas_call(
        body,
        out_shape=jax.ShapeDtypeStruct((G, D), jnp.float32),
    )(c2h, c1h, batch2d, x, W_phi, b_phi2d, W_mlp, b_mlp2d)


def kernel(x, edge_index, batch, W_phi, b_phi, W_mlp, b_mlp):
    N, D = x.shape
    E = edge_index.shape[1]
    G = _G
    sc_counts, EPT, ROWS, NC = _make_sc_counts(N, E, G)

    c2h, c1h = sc_counts(edge_index.reshape(-1), batch)
    return _tc_dense(c2h, c1h, batch.reshape(1, N), x, W_phi,
                     b_phi.reshape(1, D), W_mlp, b_mlp.reshape(1, D), G)
